# baseline (speedup 1.0000x reference)
import functools

import jax
import jax.numpy as jnp
from jax import lax
from jax.experimental import pallas as pl
from jax.experimental.pallas import tpu as pltpu
from jax.experimental.pallas import tpu_sc as plsc

N = 8192
IN_DIM = 64
H = 128
KNN = 4
HALF = N // 2
ROW_BLK = 1024
DEC_BLK = 1024


def _topk_scan(pv_ref, sqt_ref, idx_ref):
    r = ROW_BLK
    lane = 128
    inf = jnp.float32(jnp.inf)
    base_iota = lax.broadcasted_iota(jnp.int32, (r, lane), 1)
    a_v = [jnp.full((r, lane), inf, jnp.float32) for _ in range(KNN)]
    a_i = [jnp.zeros((r, lane), jnp.int32) for _ in range(KNN)]
    for j in range(N // lane):
        x = pv_ref[:, j * lane:(j + 1) * lane] + sqt_ref[:, j * lane:(j + 1) * lane]
        ix = base_iota + jnp.int32(j * lane)
        for k in range(KNN):
            c = x < a_v[k]
            nv = jnp.where(c, x, a_v[k])
            dv = jnp.where(c, a_v[k], x)
            ni = jnp.where(c, ix, a_i[k])
            di = jnp.where(c, a_i[k], ix)
            a_v[k], x, a_i[k], ix = nv, dv, ni, di
    cand_v = jnp.concatenate(a_v, axis=1)
    cand_i = jnp.concatenate(a_i, axis=1)
    cols = []
    for k in range(KNN):
        m = jnp.min(cand_v, axis=1, keepdims=True)
        am = jnp.min(jnp.where(cand_v == m, cand_i, jnp.int32(N)),
                     axis=1, keepdims=True)
        cols.append(am)
        if k + 1 < KNN:
            cand_v = jnp.where(cand_i == am, inf, cand_v)
    idx_ref[...] = jnp.concatenate(cols, axis=1)


def _enc_topk_a_body(x_ref, w1_ref, b1_ref, w2_ref, b2_ref,
                     h_out_ref, sqt_out_ref, idx_ref,
                     h_ref, sqt_ref, pv_ref):
    i = pl.program_id(0)

    @pl.when(i == 0)
    def _encode():
        h1 = jnp.maximum(jnp.dot(x_ref[...], w1_ref[...]) + b1_ref[...], 0.0)
        h = jnp.dot(h1, w2_ref[...]) + b2_ref[...]
        h_ref[...] = h
        sq = jnp.sum(h * h, axis=1, keepdims=True)
        sqt_ref[...] = lax.transpose(sq, (1, 0))

    nh = N // (HALF // ROW_BLK)
    h_out_ref[...] = h_ref[pl.ds(i * nh, nh), :]
    sqt_out_ref[...] = sqt_ref[...]
    hr = h_ref[pl.ds(i * ROW_BLK, ROW_BLK), :]
    pv_ref[...] = lax.dot_general(hr * (-2.0), h_ref[...],
                                  (((1,), (1,)), ((), ())),
                                  preferred_element_type=jnp.float32)
    _topk_scan(pv_ref, sqt_ref, idx_ref)


def _topk_b_body(hr_ref, hall_ref, sqt_ref, idx_ref, pv_ref):
    pv_ref[...] = lax.dot_general(hr_ref[...] * (-2.0), hall_ref[...],
                                  (((1,), (1,)), ((), ())),
                                  preferred_element_type=jnp.float32)
    _topk_scan(pv_ref, sqt_ref, idx_ref)


def _decoder_body(z_ref, w3_ref, b3_ref, w4_ref, b4_ref, y1_ref, y2_ref):
    zz = z_ref[...].reshape(DEC_BLK, KNN, H)
    z0 = zz[:, 0, :]
    z1 = zz[:, 1, :]
    z2 = zz[:, 2, :]
    z3 = zz[:, 3, :]
    mu = (z0 + z1 + z2 + z3) * 0.25
    mx = jnp.maximum(jnp.maximum(z0, z1), jnp.maximum(z2, z3))
    zc = jnp.concatenate([mu, mx], axis=1)
    a1 = jnp.maximum(jnp.dot(zc, w3_ref[...]) + b3_ref[...], 0.0)
    zo = jnp.dot(a1, w4_ref[...]) + b4_ref[...]
    y1_ref[...] = zo[:, :H // 2]
    y2_ref[...] = zo[:, H // 2:]


def _sc_gather(h, idx_flat):
    info = plsc.get_sparse_core_info()
    nc, ns = info.num_cores, info.num_subcores
    nw = nc * ns
    b = idx_flat.shape[0]
    b_per_w = b // nw
    ch = min(b_per_w, 256)
    nch = b_per_w // ch
    mesh = plsc.VectorSubcoreMesh(core_axis_name="c", subcore_axis_name="s")

    @functools.partial(
        pl.kernel, mesh=mesh,
        out_type=jax.ShapeDtypeStruct((b, H), jnp.float32),
        scratch_types=[
            pltpu.VMEM((ch,), jnp.int32),
            pltpu.VMEM((ch,), jnp.int32),
            pltpu.VMEM((ch, H), jnp.float32),
            pltpu.VMEM((ch, H), jnp.float32),
            pltpu.SemaphoreType.DMA,
            pltpu.SemaphoreType.DMA,
            pltpu.SemaphoreType.DMA,
            pltpu.SemaphoreType.DMA,
        ],
    )
    def gather_k(h_hbm, idx_hbm, out_hbm,
                 idx_v0, idx_v1, rows_v0, rows_v1, gs0, gs1, os0, os1):
        wid = lax.axis_index("s") * nc + lax.axis_index("c")
        idx_bufs = [idx_v0, idx_v1]
        row_bufs = [rows_v0, rows_v1]
        gsems = [gs0, gs1]
        osems = [os0, os1]

        def base(c):
            return wid * b_per_w + c * ch

        gh = [None] * nch
        oh = [None] * nch
        pltpu.sync_copy(idx_hbm.at[pl.ds(base(0), ch)], idx_bufs[0])
        gh[0] = pltpu.async_copy(h_hbm.at[idx_bufs[0]], row_bufs[0], gsems[0])
        for c in range(nch):
            cb = c & 1
            nb = 1 - cb
            if c + 1 < nch:
                pltpu.sync_copy(idx_hbm.at[pl.ds(base(c + 1), ch)],
                                idx_bufs[nb])
                if c >= 1:
                    oh[c - 1].wait()
                gh[c + 1] = pltpu.async_copy(h_hbm.at[idx_bufs[nb]],
                                             row_bufs[nb], gsems[nb])
            gh[c].wait()
            oh[c] = pltpu.async_copy(row_bufs[cb],
                                     out_hbm.at[pl.ds(base(c), ch)],
                                     osems[cb])
        if nch >= 2:
            oh[nch - 2].wait()
        oh[nch - 1].wait()

    return gather_k(h, idx_flat)


def _decode(z, W3, b3, W4, b4, rows):
    return pl.pallas_call(
        _decoder_body,
        grid=(rows // DEC_BLK,),
        in_specs=[
            pl.BlockSpec((KNN * DEC_BLK, H), lambda i: (i, 0)),
            pl.BlockSpec((2 * H, 2 * H), lambda i: (0, 0)),
            pl.BlockSpec((1, 2 * H), lambda i: (0, 0)),
            pl.BlockSpec((2 * H, H), lambda i: (0, 0)),
            pl.BlockSpec((1, H), lambda i: (0, 0)),
        ],
        out_specs=(
            pl.BlockSpec((DEC_BLK, H // 2), lambda i: (i, 0)),
            pl.BlockSpec((DEC_BLK, H // 2), lambda i: (i, 0)),
        ),
        out_shape=(
            jax.ShapeDtypeStruct((rows, H // 2), jnp.float32),
            jax.ShapeDtypeStruct((rows, H // 2), jnp.float32),
        ),
    )(z, W3, b3.reshape(1, 2 * H), W4, b4.reshape(1, H))


def kernel(x, W1, b1, W2, b2, W3, b3, W4, b4):
    h, sqt, idx_a = pl.pallas_call(
        _enc_topk_a_body,
        grid=(HALF // ROW_BLK,),
        in_specs=[
            pl.BlockSpec((N, IN_DIM), lambda i: (0, 0)),
            pl.BlockSpec((IN_DIM, H), lambda i: (0, 0)),
            pl.BlockSpec((1, H), lambda i: (0, 0)),
            pl.BlockSpec((H, H), lambda i: (0, 0)),
            pl.BlockSpec((1, H), lambda i: (0, 0)),
        ],
        out_specs=(
            pl.BlockSpec((N // (HALF // ROW_BLK), H), lambda i: (i, 0)),
            pl.BlockSpec((1, N), lambda i: (0, 0)),
            pl.BlockSpec((ROW_BLK, KNN), lambda i: (i, 0)),
        ),
        out_shape=(
            jax.ShapeDtypeStruct((N, H), jnp.float32),
            jax.ShapeDtypeStruct((1, N), jnp.float32),
            jax.ShapeDtypeStruct((HALF, KNN), jnp.int32),
        ),
        scratch_shapes=[
            pltpu.VMEM((N, H), jnp.float32),
            pltpu.VMEM((1, N), jnp.float32),
            pltpu.VMEM((ROW_BLK, N), jnp.float32),
        ],
    )(x, W1, b1.reshape(1, H), W2, b2.reshape(1, H))

    z_a = _sc_gather(h, idx_a.reshape(HALF * KNN))

    idx_b = pl.pallas_call(
        _topk_b_body,
        grid=(HALF // ROW_BLK,),
        in_specs=[
            pl.BlockSpec((ROW_BLK, H), lambda i: (i + HALF // ROW_BLK, 0)),
            pl.BlockSpec((N, H), lambda i: (0, 0)),
            pl.BlockSpec((1, N), lambda i: (0, 0)),
        ],
        out_specs=pl.BlockSpec((ROW_BLK, KNN), lambda i: (i, 0)),
        out_shape=jax.ShapeDtypeStruct((HALF, KNN), jnp.int32),
        scratch_shapes=[pltpu.VMEM((ROW_BLK, N), jnp.float32)],
    )(h, h, sqt)

    z_b = _sc_gather(h, idx_b.reshape(HALF * KNN))

    y1a, y2a = _decode(z_a, W3, b3, W4, b4, HALF)
    y1b, y2b = _decode(z_b, W3, b3, W4, b4, HALF)

    y1 = jnp.concatenate([y1a, y1b], axis=0)
    y2 = jnp.concatenate([y2a, y2b], axis=0)
    idx = jnp.concatenate([idx_a, idx_b], axis=0)
    return (y1, y2, idx)


# speedup vs baseline: 1.1128x; 1.1128x over previous
import functools

import jax
import jax.numpy as jnp
from jax import lax
from jax.experimental import pallas as pl
from jax.experimental.pallas import tpu as pltpu
from jax.experimental.pallas import tpu_sc as plsc

N = 8192
IN_DIM = 64
H = 128
KNN = 4
HALF = N // 2
ROW_BLK = 1024
DEC_BLK = 1024


def _topk_scan(pv_ref, sqt_ref, idx_ref):
    r = ROW_BLK
    lane = 128
    inf = jnp.float32(jnp.inf)
    base_iota = lax.broadcasted_iota(jnp.int32, (r, lane), 1)
    a_v = [jnp.full((r, lane), inf, jnp.float32) for _ in range(KNN)]
    a_i = [jnp.zeros((r, lane), jnp.int32) for _ in range(KNN)]
    for j in range(N // lane):
        x = pv_ref[:, j * lane:(j + 1) * lane] + sqt_ref[:, j * lane:(j + 1) * lane]
        ix = base_iota + jnp.int32(j * lane)
        for k in range(KNN):
            c = x < a_v[k]
            nv = jnp.where(c, x, a_v[k])
            dv = jnp.where(c, a_v[k], x)
            ni = jnp.where(c, ix, a_i[k])
            di = jnp.where(c, a_i[k], ix)
            a_v[k], x, a_i[k], ix = nv, dv, ni, di
    cand_v = jnp.concatenate(a_v, axis=1)
    cand_i = jnp.concatenate(a_i, axis=1)
    cols = []
    for k in range(KNN):
        m = jnp.min(cand_v, axis=1, keepdims=True)
        am = jnp.min(jnp.where(cand_v == m, cand_i, jnp.int32(N)),
                     axis=1, keepdims=True)
        cols.append(am)
        if k + 1 < KNN:
            cand_v = jnp.where(cand_i == am, inf, cand_v)
    idx_ref[...] = jnp.concatenate(cols, axis=1)


def _enc_topk_body(x_ref, w1_ref, b1_ref, w2_ref, b2_ref,
                   h_out_ref, idx_ref, h_ref, sqt_ref, pv_ref):
    i = pl.program_id(0)

    @pl.when(i == 0)
    def _encode():
        h1 = jnp.maximum(jnp.dot(x_ref[...], w1_ref[...]) + b1_ref[...], 0.0)
        h = jnp.dot(h1, w2_ref[...]) + b2_ref[...]
        h_ref[...] = h
        sq = jnp.sum(h * h, axis=1, keepdims=True)
        sqt_ref[...] = lax.transpose(sq, (1, 0))

    hr = h_ref[pl.ds(i * ROW_BLK, ROW_BLK), :]
    h_out_ref[...] = hr
    pv_ref[...] = lax.dot_general(hr * (-2.0), h_ref[...],
                                  (((1,), (1,)), ((), ())),
                                  preferred_element_type=jnp.float32)
    _topk_scan(pv_ref, sqt_ref, idx_ref)


def _decoder_body(z_ref, w3_ref, b3_ref, w4_ref, b4_ref, y1_ref, y2_ref):
    zz = z_ref[...].reshape(DEC_BLK, KNN, H)
    z0 = zz[:, 0, :]
    z1 = zz[:, 1, :]
    z2 = zz[:, 2, :]
    z3 = zz[:, 3, :]
    mu = (z0 + z1 + z2 + z3) * 0.25
    mx = jnp.maximum(jnp.maximum(z0, z1), jnp.maximum(z2, z3))
    zc = jnp.concatenate([mu, mx], axis=1)
    a1 = jnp.maximum(jnp.dot(zc, w3_ref[...]) + b3_ref[...], 0.0)
    zo = jnp.dot(a1, w4_ref[...]) + b4_ref[...]
    y1_ref[...] = zo[:, :H // 2]
    y2_ref[...] = zo[:, H // 2:]


def _sc_gather(h, idx_flat):
    info = plsc.get_sparse_core_info()
    nc, ns = info.num_cores, info.num_subcores
    nw = nc * ns
    b = idx_flat.shape[0]
    b_per_w = b // nw
    ch = min(b_per_w, 256)
    nch = b_per_w // ch
    mesh = plsc.VectorSubcoreMesh(core_axis_name="c", subcore_axis_name="s")

    @functools.partial(
        pl.kernel, mesh=mesh,
        out_type=jax.ShapeDtypeStruct((b, H), jnp.float32),
        scratch_types=[
            pltpu.VMEM((ch,), jnp.int32),
            pltpu.VMEM((ch,), jnp.int32),
            pltpu.VMEM((ch, H), jnp.float32),
            pltpu.VMEM((ch, H), jnp.float32),
            pltpu.SemaphoreType.DMA,
            pltpu.SemaphoreType.DMA,
            pltpu.SemaphoreType.DMA,
            pltpu.SemaphoreType.DMA,
        ],
    )
    def gather_k(h_hbm, idx_hbm, out_hbm,
                 idx_v0, idx_v1, rows_v0, rows_v1, gs0, gs1, os0, os1):
        wid = lax.axis_index("s") * nc + lax.axis_index("c")
        idx_bufs = [idx_v0, idx_v1]
        row_bufs = [rows_v0, rows_v1]
        gsems = [gs0, gs1]
        osems = [os0, os1]

        def base(c):
            return wid * b_per_w + c * ch

        gh = [None] * nch
        oh = [None] * nch
        pltpu.sync_copy(idx_hbm.at[pl.ds(base(0), ch)], idx_bufs[0])
        gh[0] = pltpu.async_copy(h_hbm.at[idx_bufs[0]], row_bufs[0], gsems[0])
        for c in range(nch):
            cb = c & 1
            nb = 1 - cb
            if c + 1 < nch:
                pltpu.sync_copy(idx_hbm.at[pl.ds(base(c + 1), ch)],
                                idx_bufs[nb])
                if c >= 1:
                    oh[c - 1].wait()
                gh[c + 1] = pltpu.async_copy(h_hbm.at[idx_bufs[nb]],
                                             row_bufs[nb], gsems[nb])
            gh[c].wait()
            oh[c] = pltpu.async_copy(row_bufs[cb],
                                     out_hbm.at[pl.ds(base(c), ch)],
                                     osems[cb])
        if nch >= 2:
            oh[nch - 2].wait()
        oh[nch - 1].wait()

    return gather_k(h, idx_flat)


def _decode(z, W3, b3, W4, b4, rows):
    return pl.pallas_call(
        _decoder_body,
        grid=(rows // DEC_BLK,),
        in_specs=[
            pl.BlockSpec((KNN * DEC_BLK, H), lambda i: (i, 0)),
            pl.BlockSpec((2 * H, 2 * H), lambda i: (0, 0)),
            pl.BlockSpec((1, 2 * H), lambda i: (0, 0)),
            pl.BlockSpec((2 * H, H), lambda i: (0, 0)),
            pl.BlockSpec((1, H), lambda i: (0, 0)),
        ],
        out_specs=(
            pl.BlockSpec((DEC_BLK, H // 2), lambda i: (i, 0)),
            pl.BlockSpec((DEC_BLK, H // 2), lambda i: (i, 0)),
        ),
        out_shape=(
            jax.ShapeDtypeStruct((rows, H // 2), jnp.float32),
            jax.ShapeDtypeStruct((rows, H // 2), jnp.float32),
        ),
    )(z, W3, b3.reshape(1, 2 * H), W4, b4.reshape(1, H))


def kernel(x, W1, b1, W2, b2, W3, b3, W4, b4):
    h, idx = pl.pallas_call(
        _enc_topk_body,
        grid=(N // ROW_BLK,),
        in_specs=[
            pl.BlockSpec((N, IN_DIM), lambda i: (0, 0)),
            pl.BlockSpec((IN_DIM, H), lambda i: (0, 0)),
            pl.BlockSpec((1, H), lambda i: (0, 0)),
            pl.BlockSpec((H, H), lambda i: (0, 0)),
            pl.BlockSpec((1, H), lambda i: (0, 0)),
        ],
        out_specs=(
            pl.BlockSpec((ROW_BLK, H), lambda i: (i, 0)),
            pl.BlockSpec((ROW_BLK, KNN), lambda i: (i, 0)),
        ),
        out_shape=(
            jax.ShapeDtypeStruct((N, H), jnp.float32),
            jax.ShapeDtypeStruct((N, KNN), jnp.int32),
        ),
        scratch_shapes=[
            pltpu.VMEM((N, H), jnp.float32),
            pltpu.VMEM((1, N), jnp.float32),
            pltpu.VMEM((ROW_BLK, N), jnp.float32),
        ],
    )(x, W1, b1.reshape(1, H), W2, b2.reshape(1, H))

    z = _sc_gather(h, idx.reshape(N * KNN))
    y1, y2 = _decode(z, W3, b3, W4, b4, N)
    return (y1, y2, idx)
